# baseline (speedup 1.0000x reference)
import functools

import jax
import jax.numpy as jnp
from jax import lax
from jax.experimental import pallas as pl
from jax.experimental.pallas import tpu as pltpu
from jax.experimental.pallas import tpu_sc as plsc

NUM_FIELDS = 26
FIELD_DIM = 40000
BATCH = 16384
NC = 2
NS = 16
B_PER_CORE = BATCH // NC
B_PER_TILE = B_PER_CORE // NS
L = 16


def _sc_body(x, table, bias, out, xblk_v, colbuf_v, tab_v, idx_v,
             part_v, tmp_v, out_v, bias_v, shared_x, shared_p):
    c = lax.axis_index("c")
    s = lax.axis_index("s")
    base_b = c * B_PER_CORE

    pltpu.sync_copy(bias, bias_v)

    blk0 = pl.multiple_of((base_b + s * B_PER_TILE) * NUM_FIELDS,
                          B_PER_TILE * NUM_FIELDS)
    pltpu.sync_copy(x.at[pl.ds(blk0, B_PER_TILE * NUM_FIELDS)], xblk_v)

    iota26 = lax.iota(jnp.int32, L) * NUM_FIELDS

    for f in range(NUM_FIELDS):

        @pl.loop(0, B_PER_TILE // L)
        def _extract(j):
            flat = iota26 + (j * (L * NUM_FIELDS) + f)
            colbuf_v[pl.ds(j * L, L)] = plsc.load_gather(xblk_v, [flat])

        pltpu.sync_copy(colbuf_v,
                        shared_x.at[pl.ds(f * B_PER_CORE + s * B_PER_TILE,
                                          B_PER_TILE)])
    plsc.subcore_barrier()

    def _load_field_table(f):
        pltpu.sync_copy(table.at[pl.ds(f * FIELD_DIM, FIELD_DIM)], tab_v)

    f1 = s
    pltpu.sync_copy(shared_x.at[pl.ds(f1 * B_PER_CORE, B_PER_CORE)], idx_v)
    _load_field_table(f1)

    @pl.loop(0, B_PER_CORE // L)
    def _gather1(j):
        sl = pl.ds(j * L, L)
        part_v[sl] = plsc.load_gather(tab_v, [idx_v[sl]])

    @pl.when(s < NUM_FIELDS - NS)
    def _second_field():
        f2 = s + NS
        pltpu.sync_copy(shared_x.at[pl.ds(f2 * B_PER_CORE, B_PER_CORE)],
                        idx_v)
        _load_field_table(f2)

        @pl.loop(0, B_PER_CORE // L)
        def _gather2(j):
            sl = pl.ds(j * L, L)
            part_v[sl] = part_v[sl] + plsc.load_gather(tab_v, [idx_v[sl]])

    pltpu.sync_copy(part_v, shared_p.at[pl.ds(s * B_PER_CORE, B_PER_CORE)])
    plsc.subcore_barrier()

    bias_vec = bias_v[...]

    @pl.loop(0, B_PER_TILE // L)
    def _init(j):
        out_v[pl.ds(j * L, L)] = bias_vec

    for t in range(NS):
        pltpu.sync_copy(
            shared_p.at[pl.ds(t * B_PER_CORE + s * B_PER_TILE, B_PER_TILE)],
            tmp_v)

        @pl.loop(0, B_PER_TILE // L)
        def _acc(j):
            sl = pl.ds(j * L, L)
            out_v[sl] = out_v[sl] + tmp_v[sl]

    pltpu.sync_copy(out_v, out.at[pl.ds(base_b + s * B_PER_TILE, B_PER_TILE)])


_sc_kernel = functools.partial(
    pl.kernel,
    out_type=jax.ShapeDtypeStruct((BATCH,), jnp.float32),
    mesh=plsc.VectorSubcoreMesh(core_axis_name="c", subcore_axis_name="s",
                                num_cores=NC, num_subcores=NS),
    scratch_types=[
        pltpu.VMEM((B_PER_TILE * NUM_FIELDS,), jnp.int32),
        pltpu.VMEM((B_PER_TILE,), jnp.int32),
        pltpu.VMEM((FIELD_DIM,), jnp.float32),
        pltpu.VMEM((B_PER_CORE,), jnp.int32),
        pltpu.VMEM((B_PER_CORE,), jnp.float32),
        pltpu.VMEM((B_PER_TILE,), jnp.float32),
        pltpu.VMEM((B_PER_TILE,), jnp.float32),
        pltpu.VMEM((L,), jnp.float32),
        pltpu.VMEM_SHARED((NUM_FIELDS * B_PER_CORE,), jnp.int32),
        pltpu.VMEM_SHARED((NS * B_PER_CORE,), jnp.float32),
    ],
    compiler_params=pltpu.CompilerParams(needs_layout_passes=False),
)(_sc_body)


@jax.jit
def kernel(x, fc_weight, bias):
    xi = x.astype(jnp.int32).reshape(-1)
    table = fc_weight.reshape(-1).astype(jnp.float32)
    bias16 = jnp.broadcast_to(bias.astype(jnp.float32), (L,))
    out = _sc_kernel(xi, table, bias16)
    return out.reshape(BATCH, 1)


# speedup vs baseline: 2.2453x; 2.2453x over previous
import functools

import jax
import jax.numpy as jnp
from jax import lax
from jax.experimental import pallas as pl
from jax.experimental.pallas import tpu as pltpu
from jax.experimental.pallas import tpu_sc as plsc

NUM_FIELDS = 26
FIELD_DIM = 40000
BATCH = 16384
NC = 2
NS = 16
B_PER_CORE = BATCH // NC
B_PER_TILE = B_PER_CORE // NS
L = 16
TAB_NROW = 8128
TAB_ROWS = 320
TAB_CAP_ROW = TAB_NROW - TAB_ROWS


def _sc_body(xT, table, bias, out, tab_v, idx_v, part_v, tmp_v, out_v,
             bias_v, shared_p):
    c = lax.axis_index("c")
    s = lax.axis_index("s")
    base_b = c * B_PER_CORE

    pltpu.sync_copy(bias, bias_v)

    def _do_field(f, first):
        pltpu.sync_copy(xT.at[f, pl.ds(base_b, B_PER_CORE)], idx_v)
        flat0 = f * FIELD_DIM
        row_start = jnp.minimum((flat0 >> 10) << 3, TAB_CAP_ROW)
        row_start = pl.multiple_of(row_start, 8)
        rem = flat0 - (row_start << 7)
        pltpu.sync_copy(table.at[pl.ds(row_start, TAB_ROWS)], tab_v)

        if first:
            @pl.loop(0, B_PER_CORE // L)
            def _gather(j):
                sl = pl.ds(j * L, L)
                w = idx_v[sl] + rem
                part_v[sl] = plsc.load_gather(tab_v, [w >> 7, w & 127])
        else:
            @pl.loop(0, B_PER_CORE // L)
            def _gather(j):
                sl = pl.ds(j * L, L)
                w = idx_v[sl] + rem
                part_v[sl] = part_v[sl] + plsc.load_gather(tab_v,
                                                           [w >> 7, w & 127])

    _do_field(s, True)

    @pl.when(s < NUM_FIELDS - NS)
    def _second_field():
        _do_field(s + NS, False)

    pltpu.sync_copy(part_v, shared_p.at[pl.ds(s * B_PER_CORE, B_PER_CORE)])
    plsc.subcore_barrier()

    bias_vec = bias_v[...]

    @pl.loop(0, B_PER_TILE // L)
    def _init(j):
        out_v[pl.ds(j * L, L)] = bias_vec

    for t in range(NS):
        pltpu.sync_copy(
            shared_p.at[pl.ds(t * B_PER_CORE + s * B_PER_TILE, B_PER_TILE)],
            tmp_v)

        @pl.loop(0, B_PER_TILE // L)
        def _acc(j):
            sl = pl.ds(j * L, L)
            out_v[sl] = out_v[sl] + tmp_v[sl]

    pltpu.sync_copy(out_v, out.at[pl.ds(base_b + s * B_PER_TILE, B_PER_TILE)])


_sc_kernel = functools.partial(
    pl.kernel,
    out_type=jax.ShapeDtypeStruct((BATCH,), jnp.float32),
    mesh=plsc.VectorSubcoreMesh(core_axis_name="c", subcore_axis_name="s",
                                num_cores=NC, num_subcores=NS),
    scratch_types=[
        pltpu.VMEM((TAB_ROWS, 128), jnp.float32),
        pltpu.VMEM((B_PER_CORE,), jnp.int32),
        pltpu.VMEM((B_PER_CORE,), jnp.float32),
        pltpu.VMEM((B_PER_TILE,), jnp.float32),
        pltpu.VMEM((B_PER_TILE,), jnp.float32),
        pltpu.VMEM((L,), jnp.float32),
        pltpu.VMEM_SHARED((NS * B_PER_CORE,), jnp.float32),
    ],
    compiler_params=pltpu.CompilerParams(needs_layout_passes=False),
)(_sc_body)


@jax.jit
def kernel(x, fc_weight, bias):
    xT = x.astype(jnp.int32).T
    fcp = jnp.pad(fc_weight.astype(jnp.float32), ((0, 384), (0, 0)))
    table = fcp.reshape(TAB_NROW, 128)
    bias16 = jnp.broadcast_to(bias.astype(jnp.float32), (L,))
    out = _sc_kernel(xT, table, bias16)
    return out.reshape(BATCH, 1)
